# Initial kernel scaffold; baseline (speedup 1.0000x reference)
import functools

import jax
import jax.numpy as jnp
from jax import lax
from jax.experimental import pallas as pl
from jax.experimental.pallas import tpu as pltpu
from jax.experimental.pallas import tpu_sc as plsc

N = 320000
D = 128
S = 3200
NC = 2
NS = 16
NW = NC * NS
GROUP = 128
G = N // GROUP
ROWS_PER_TILE = S // NS


def _sc_body(x_hbm, seg_hbm, y_hbm, psum_hbm, pyc_hbm,
             xbuf, ohbuf, segbuf, ybuf, ssum, syc):
  cid = lax.axis_index("c")
  sid = lax.axis_index("s")
  wid = cid * NS + sid

  zeros16 = jnp.zeros((16,), jnp.float32)
  ones16 = jnp.ones((16,), jnp.float32)
  lane = lax.iota(jnp.int32, 16)

  def zrow(r, _):
    for c in range(D // 16):
      xbuf[r, pl.ds(c * 16, 16)] = zeros16
    ohbuf[r, :] = zeros16
    return 0
  lax.fori_loop(0, GROUP, zrow, 0)

  base = sid * ROWS_PER_TILE
  pltpu.sync_copy(xbuf.at[pl.ds(0, 128)], ssum.at[pl.ds(base, 128)])
  pltpu.sync_copy(xbuf.at[pl.ds(0, 72)], ssum.at[pl.ds(base + 128, 72)])
  pltpu.sync_copy(ohbuf.at[pl.ds(0, 128)], syc.at[pl.ds(base, 128)])
  pltpu.sync_copy(ohbuf.at[pl.ds(0, 72)], syc.at[pl.ds(base + 128, 72)])
  plsc.subcore_barrier()

  g_lo = wid * G // NW
  g_hi = (wid + 1) * G // NW

  def group_body(g, _):
    pltpu.sync_copy(x_hbm.at[pl.ds(g * GROUP, GROUP)], xbuf)
    pltpu.sync_copy(seg_hbm.at[g], segbuf)
    pltpu.sync_copy(y_hbm.at[g], ybuf)
    for k in range(GROUP // 16):
      yv = ybuf[pl.ds(k * 16, 16)]
      plsc.store_scatter(ohbuf, [lane + k * 16, yv], ones16)
    pltpu.sync_copy(xbuf, ssum.at[segbuf], add=True)
    pltpu.sync_copy(ohbuf, syc.at[segbuf], add=True)
    for k in range(GROUP // 16):
      yv = ybuf[pl.ds(k * 16, 16)]
      plsc.store_scatter(ohbuf, [lane + k * 16, yv], zeros16)
    return 0
  lax.fori_loop(g_lo, g_hi, group_body, 0)

  plsc.subcore_barrier()
  pltpu.sync_copy(ssum.at[pl.ds(base, ROWS_PER_TILE)],
                  psum_hbm.at[cid, pl.ds(base, ROWS_PER_TILE)])
  pltpu.sync_copy(syc.at[pl.ds(base, ROWS_PER_TILE)],
                  pyc_hbm.at[cid, pl.ds(base, ROWS_PER_TILE)])


_sc_call = pl.kernel(
    _sc_body,
    out_type=(
        jax.ShapeDtypeStruct((NC, S, D), jnp.float32),
        jax.ShapeDtypeStruct((NC, S, 16), jnp.float32),
    ),
    mesh=plsc.VectorSubcoreMesh(core_axis_name="c", subcore_axis_name="s"),
    scratch_types=[
        pltpu.VMEM((GROUP, D), jnp.float32),
        pltpu.VMEM((GROUP, 16), jnp.float32),
        pltpu.VMEM((GROUP,), jnp.int32),
        pltpu.VMEM((GROUP,), jnp.int32),
        pltpu.VMEM_SHARED((S, D), jnp.float32),
        pltpu.VMEM_SHARED((S, 16), jnp.float32),
    ],
)


def _fin_body(ps_ref, pyc_ref, xs_ref, ys_ref, m_ref):
  sums = ps_ref[0] + ps_ref[1]
  yc = pyc_ref[0] + pyc_ref[1]
  counts = jnp.sum(yc, axis=1)
  xs_ref[...] = sums / jnp.maximum(counts, 1.0)[:, None]
  mx = jnp.max(yc, axis=1, keepdims=True)
  lane = lax.broadcasted_iota(jnp.int32, (S, 16), 1)
  idx = jnp.min(jnp.where(yc >= mx, lane, 16), axis=1)
  ys_ref[...] = jnp.where(counts > 0, idx, -1)
  m_ref[...] = (counts > 0).astype(jnp.int32)


_fin_call = pl.pallas_call(
    _fin_body,
    out_shape=(
        jax.ShapeDtypeStruct((S, D), jnp.float32),
        jax.ShapeDtypeStruct((S,), jnp.int32),
        jax.ShapeDtypeStruct((S,), jnp.int32),
    ),
)


def kernel(x, segment_ids, y):
  seg2 = segment_ids.astype(jnp.int32).reshape(G, GROUP)
  y2 = y.astype(jnp.int32).reshape(G, GROUP)
  psum, pyc = _sc_call(x, seg2, y2)
  x_syn, y_syn, m = _fin_call(psum, pyc)
  return (x_syn, y_syn, m != 0)


# speedup vs baseline: 5.8663x; 5.8663x over previous
import jax
import jax.numpy as jnp
from jax import lax
from jax.experimental import pallas as pl
from jax.experimental.pallas import tpu as pltpu
from jax.experimental.pallas import tpu_sc as plsc

N = 320000
D = 128
S = 3200
C = 8
NC = 2
NS = 16
NW = NC * NS
GROUP = 128
G = N // GROUP
ROWS_PER_TILE = S // NS


def _sc_body(x_hbm, seg_hbm, y_hbm, psum_hbm, pyc_hbm,
             xbuf, obuf, segbuf, ybuf, ssum, syc):
  cid = lax.axis_index("c")
  sid = lax.axis_index("s")
  wid = cid * NS + sid

  zeros16 = jnp.zeros((16,), jnp.float32)
  e1 = jnp.where(lax.iota(jnp.int32, 16) == 0, 1.0, 0.0).astype(jnp.float32)

  def zrow(r, _):
    for c in range(D // 16):
      xbuf[r, pl.ds(c * 16, 16)] = zeros16
      obuf[r, pl.ds(c * 16, 16)] = zeros16
    return 0
  lax.fori_loop(0, GROUP, zrow, 0)

  base = sid * ROWS_PER_TILE
  pltpu.sync_copy(xbuf.at[pl.ds(0, 128)], ssum.at[pl.ds(base, 128)])
  pltpu.sync_copy(xbuf.at[pl.ds(0, 72)], ssum.at[pl.ds(base + 128, 72)])
  pltpu.sync_copy(xbuf.at[pl.ds(0, 128)], syc.at[pl.ds(base, 128)])
  pltpu.sync_copy(xbuf.at[pl.ds(0, 72)], syc.at[pl.ds(base + 128, 72)])
  plsc.subcore_barrier()

  g_lo = wid * G // NW
  g_hi = (wid + 1) * G // NW

  def group_body(g, _):
    pltpu.sync_copy(x_hbm.at[pl.ds(g * GROUP, GROUP)], xbuf)
    pltpu.sync_copy(seg_hbm.at[g], segbuf)
    pltpu.sync_copy(y_hbm.at[g], ybuf)
    for k in range(GROUP // 16):
      yv = ybuf[pl.ds(k * 16, 16)]
      for j in range(16):
        obuf[k * 16 + j, pl.ds(yv[j] * 16, 16)] = e1
    pltpu.sync_copy(xbuf, ssum.at[segbuf], add=True)
    pltpu.sync_copy(obuf, syc.at[segbuf], add=True)
    for k in range(GROUP // 16):
      yv = ybuf[pl.ds(k * 16, 16)]
      for j in range(16):
        obuf[k * 16 + j, pl.ds(yv[j] * 16, 16)] = zeros16
    return 0
  lax.fori_loop(g_lo, g_hi, group_body, 0)

  plsc.subcore_barrier()
  pltpu.sync_copy(ssum.at[pl.ds(base, ROWS_PER_TILE)],
                  psum_hbm.at[cid, pl.ds(base, ROWS_PER_TILE)])
  pltpu.sync_copy(syc.at[pl.ds(base, ROWS_PER_TILE)],
                  pyc_hbm.at[cid, pl.ds(base, ROWS_PER_TILE)])


_sc_call = pl.kernel(
    _sc_body,
    out_type=(
        jax.ShapeDtypeStruct((NC, S, D), jnp.float32),
        jax.ShapeDtypeStruct((NC, S, D), jnp.float32),
    ),
    mesh=plsc.VectorSubcoreMesh(core_axis_name="c", subcore_axis_name="s"),
    scratch_types=[
        pltpu.VMEM((GROUP, D), jnp.float32),
        pltpu.VMEM((GROUP, D), jnp.float32),
        pltpu.VMEM((GROUP,), jnp.int32),
        pltpu.VMEM((GROUP,), jnp.int32),
        pltpu.VMEM_SHARED((S, D), jnp.float32),
        pltpu.VMEM_SHARED((S, D), jnp.float32),
    ],
)


def _fin_body(ps_ref, pyc_ref, xs_ref, ys_ref, m_ref):
  sums = ps_ref[0] + ps_ref[1]
  yc = pyc_ref[0] + pyc_ref[1]
  counts = jnp.sum(yc, axis=1)
  xs_ref[...] = sums / jnp.maximum(counts, 1.0)[:, None]
  mx = jnp.max(yc, axis=1, keepdims=True)
  lane = lax.broadcasted_iota(jnp.int32, (S, D), 1)
  idx = jnp.min(jnp.where(yc >= mx, lane, D), axis=1)
  ys_ref[...] = jnp.where(counts > 0, idx // 16, -1)
  m_ref[...] = (counts > 0).astype(jnp.int32)


_fin_call = pl.pallas_call(
    _fin_body,
    out_shape=(
        jax.ShapeDtypeStruct((S, D), jnp.float32),
        jax.ShapeDtypeStruct((S,), jnp.int32),
        jax.ShapeDtypeStruct((S,), jnp.int32),
    ),
)


def kernel(x, segment_ids, y):
  seg2 = segment_ids.astype(jnp.int32).reshape(G, GROUP)
  y2 = y.astype(jnp.int32).reshape(G, GROUP)
  psum, pyc = _sc_call(x, seg2, y2)
  x_syn, y_syn, m = _fin_call(psum, pyc)
  return (x_syn, y_syn, m != 0)


# speedup vs baseline: 9.4163x; 1.6051x over previous
import jax
import jax.numpy as jnp
from jax import lax
from jax.experimental import pallas as pl
from jax.experimental.pallas import tpu as pltpu
from jax.experimental.pallas import tpu_sc as plsc

N = 320000
D = 128
S = 3200
C = 8
NC = 2
NS = 16
NW = NC * NS
GROUP = 128
G = N // GROUP
ROWS_PER_TILE = S // NS


def _sc_body(x_hbm, seg_hbm, y_hbm, psum_hbm, pyc_hbm,
             xbuf, obuf, segbuf, ybuf, ssum, syc, lsem, ssem):
  cid = lax.axis_index("c")
  sid = lax.axis_index("s")
  wid = cid * NS + sid

  zeros16 = jnp.zeros((16,), jnp.float32)
  e1 = jnp.where(lax.iota(jnp.int32, 16) == 0, 1.0, 0.0).astype(jnp.float32)

  def zrow(r, _):
    for c in range(D // 16):
      xbuf[0, r, pl.ds(c * 16, 16)] = zeros16
      obuf[0, r, pl.ds(c * 16, 16)] = zeros16
      obuf[1, r, pl.ds(c * 16, 16)] = zeros16
    return 0
  lax.fori_loop(0, GROUP, zrow, 0)

  base = sid * ROWS_PER_TILE
  pltpu.sync_copy(xbuf.at[0, pl.ds(0, 128)], ssum.at[pl.ds(base, 128)])
  pltpu.sync_copy(xbuf.at[0, pl.ds(0, 72)], ssum.at[pl.ds(base + 128, 72)])
  pltpu.sync_copy(xbuf.at[0, pl.ds(0, 128)], syc.at[pl.ds(base, 128)])
  pltpu.sync_copy(xbuf.at[0, pl.ds(0, 72)], syc.at[pl.ds(base + 128, 72)])
  plsc.subcore_barrier()

  g_lo = wid * G // NW
  g_hi = (wid + 1) * G // NW

  def start_loads(g, p):
    pltpu.async_copy(x_hbm.at[pl.ds(g * GROUP, GROUP)], xbuf.at[p],
                     lsem.at[p])
    pltpu.async_copy(seg_hbm.at[g], segbuf.at[p], lsem.at[p])
    pltpu.async_copy(y_hbm.at[g], ybuf.at[p], lsem.at[p])

  def wait_loads(g, p):
    pltpu.make_async_copy(x_hbm.at[pl.ds(g * GROUP, GROUP)], xbuf.at[p],
                          lsem.at[p]).wait()
    pltpu.make_async_copy(seg_hbm.at[g], segbuf.at[p], lsem.at[p]).wait()
    pltpu.make_async_copy(y_hbm.at[g], ybuf.at[p], lsem.at[p]).wait()

  def start_scatters(p):
    pltpu.async_copy(xbuf.at[p], ssum.at[segbuf.at[p]], ssem.at[p],
                     add=True)
    pltpu.async_copy(obuf.at[p], syc.at[segbuf.at[p]], ssem.at[p],
                     add=True)

  def wait_scatters(p):
    pltpu.make_async_copy(xbuf.at[p], ssum.at[segbuf.at[p]],
                          ssem.at[p]).wait()
    pltpu.make_async_copy(obuf.at[p], syc.at[segbuf.at[p]],
                          ssem.at[p]).wait()

  start_loads(g_lo, 0)

  def group_body(g, _):
    p = (g - g_lo) % 2
    q = 1 - p

    @pl.when(g > g_lo)
    def _():
      wait_scatters(q)
      for k in range(GROUP // 16):
        yv = ybuf[q, pl.ds(k * 16, 16)]
        for j in range(16):
          obuf[q, k * 16 + j, pl.ds(yv[j] * 16, 16)] = zeros16

    @pl.when(g + 1 < g_hi)
    def _():
      start_loads(g + 1, q)

    wait_loads(g, p)
    for k in range(GROUP // 16):
      yv = ybuf[p, pl.ds(k * 16, 16)]
      for j in range(16):
        obuf[p, k * 16 + j, pl.ds(yv[j] * 16, 16)] = e1
    start_scatters(p)
    return 0
  lax.fori_loop(g_lo, g_hi, group_body, 0)

  last_p = (g_hi - 1 - g_lo) % 2
  wait_scatters(last_p)

  plsc.subcore_barrier()
  pltpu.sync_copy(ssum.at[pl.ds(base, ROWS_PER_TILE)],
                  psum_hbm.at[cid, pl.ds(base, ROWS_PER_TILE)])
  pltpu.sync_copy(syc.at[pl.ds(base, ROWS_PER_TILE)],
                  pyc_hbm.at[cid, pl.ds(base, ROWS_PER_TILE)])


_sc_call = pl.kernel(
    _sc_body,
    out_type=(
        jax.ShapeDtypeStruct((NC, S, D), jnp.float32),
        jax.ShapeDtypeStruct((NC, S, D), jnp.float32),
    ),
    mesh=plsc.VectorSubcoreMesh(core_axis_name="c", subcore_axis_name="s"),
    scratch_types=[
        pltpu.VMEM((2, GROUP, D), jnp.float32),
        pltpu.VMEM((2, GROUP, D), jnp.float32),
        pltpu.VMEM((2, GROUP), jnp.int32),
        pltpu.VMEM((2, GROUP), jnp.int32),
        pltpu.VMEM_SHARED((S, D), jnp.float32),
        pltpu.VMEM_SHARED((S, D), jnp.float32),
        pltpu.SemaphoreType.DMA((2,)),
        pltpu.SemaphoreType.DMA((2,)),
    ],
)


def _fin_body(ps_ref, pyc_ref, xs_ref, ys_ref, m_ref):
  sums = ps_ref[0] + ps_ref[1]
  yc = pyc_ref[0] + pyc_ref[1]
  counts = jnp.sum(yc, axis=1)
  xs_ref[...] = sums / jnp.maximum(counts, 1.0)[:, None]
  mx = jnp.max(yc, axis=1, keepdims=True)
  lane = lax.broadcasted_iota(jnp.int32, (S, D), 1)
  idx = jnp.min(jnp.where(yc >= mx, lane, D), axis=1)
  ys_ref[...] = jnp.where(counts > 0, idx // 16, -1)
  m_ref[...] = (counts > 0).astype(jnp.int32)


_fin_call = pl.pallas_call(
    _fin_body,
    out_shape=(
        jax.ShapeDtypeStruct((S, D), jnp.float32),
        jax.ShapeDtypeStruct((S,), jnp.int32),
        jax.ShapeDtypeStruct((S,), jnp.int32),
    ),
)


def kernel(x, segment_ids, y):
  seg2 = segment_ids.astype(jnp.int32).reshape(G, GROUP)
  y2 = y.astype(jnp.int32).reshape(G, GROUP)
  psum, pyc = _sc_call(x, seg2, y2)
  x_syn, y_syn, m = _fin_call(psum, pyc)
  return (x_syn, y_syn, m != 0)
